# baseline (speedup 1.0000x reference)
import functools

import numpy as np
import jax
import jax.numpy as jnp
from jax import lax
from jax.experimental import pallas as pl
from jax.experimental.pallas import tpu as pltpu
from jax.experimental.pallas import tpu_sc as plsc

D_FEAT = 123
D_PAD = 128
NUM_VALS = 21
NC, NS = 2, 16
NW = NC * NS
BATCH, SEQ = 1024, 512
B_TOKENS = BATCH * SEQ
TOK_PER_W = B_TOKENS // NW
CHUNK = 128
NCHUNK = TOK_PER_W // CHUNK
NBUF = 4
LOOKAHEAD = 3


def _expansion_consts():
    mu = np.zeros((D_PAD,), np.float32)
    inv = np.zeros((D_PAD,), np.float32)
    ch = np.zeros((D_PAD,), np.int64)
    sig = np.zeros((D_PAD,), np.float32)
    o = 0
    for cnt, c, lo, hi, stride in (
        (90, 0, -4.5, 4.5, 0.1),
        (22, 1, 0.0, 2.2, 0.1),
        (8, 2, -1.0, 1.0, 0.25),
    ):
        mu[o:o + cnt] = np.linspace(lo, hi, cnt)
        inv[o:o + cnt] = 1.0 / stride
        ch[o:o + cnt] = c
        o += cnt
    for c in (3, 4, 5):
        ch[o] = c
        sig[o] = 1.0
        o += 1
    sel = np.zeros((6, D_PAD), np.float32)
    sel[ch[:D_FEAT], np.arange(D_FEAT)] = 1.0
    return (
        jnp.asarray(mu).reshape(1, -1),
        jnp.asarray(inv).reshape(1, -1),
        jnp.asarray(sig).reshape(1, -1),
        jnp.asarray(sel),
    )


def _table_body(emb_ref, sel_ref, mu_ref, inv_ref, sig_ref, tab_ref):
    e = jnp.zeros(tab_ref.shape, jnp.float32)
    for c in range(6):
        e = e + emb_ref[:, c:c + 1] * sel_ref[c:c + 1, :]
    rbf = jnp.exp(-((e - mu_ref[...]) * inv_ref[...]) ** 2)
    sg = jax.nn.sigmoid(e * 6.0 - 3.0)
    tab_ref[...] = jnp.where(sig_ref[...] > 0.0, sg, rbf)


_table_call = pl.pallas_call(
    _table_body,
    out_shape=jax.ShapeDtypeStruct((NUM_VALS, D_PAD), jnp.float32),
)


@functools.partial(
    pl.kernel,
    out_type=jax.ShapeDtypeStruct((B_TOKENS, D_PAD), jnp.float32),
    mesh=plsc.VectorSubcoreMesh(core_axis_name="c", subcore_axis_name="s"),
    scratch_types=[
        pltpu.VMEM((TOK_PER_W,), jnp.int32),
        pltpu.VMEM_SHARED((NUM_VALS, D_PAD), jnp.float32),
        pltpu.VMEM((NBUF, CHUNK, D_PAD), jnp.float32),
        pltpu.SemaphoreType.DMA,
        pltpu.SemaphoreType.DMA,
        pltpu.SemaphoreType.DMA,
    ],
)
def _gather_call(tab_hbm, idx_hbm, out_hbm, idx_v, tab_v, bufs, tsem, gsem, wsem):
    sid = lax.axis_index("s")
    wid = sid * NC + lax.axis_index("c")
    base = wid * TOK_PER_W

    @pl.when(sid == 0)
    def _stage_table():
        pltpu.make_async_copy(tab_hbm, tab_v, tsem).start()

    pltpu.sync_copy(idx_hbm.at[pl.ds(base, TOK_PER_W)], idx_v)

    @pl.when(sid == 0)
    def _stage_table_wait():
        pltpu.make_async_copy(tab_hbm, tab_v, tsem).wait()

    plsc.subcore_barrier()

    def g_desc(c, b):
        return pltpu.make_async_copy(
            tab_v.at[idx_v.at[pl.ds(c * CHUNK, CHUNK)]], bufs.at[b], gsem)

    def w_desc(c, b):
        return pltpu.make_async_copy(
            bufs.at[b], out_hbm.at[pl.ds(base + c * CHUNK, CHUNK)], wsem)

    for c in range(LOOKAHEAD):
        g_desc(c, c % NBUF).start()

    @pl.loop(0, NCHUNK, step=NBUF)
    def _ring(c4):
        for d in range(NBUF):
            c = c4 + d
            g_desc(c, d).wait()
            w_desc(c, d).start()
            prev = c - 1
            if d == 0:
                @pl.when(c4 > 0)
                def _w():
                    w_desc(prev, NBUF - 1).wait()
            else:
                w_desc(prev, d - 1).wait()
            nxt = c + LOOKAHEAD
            nb = (d + LOOKAHEAD) % NBUF

            @pl.when(nxt < NCHUNK)
            def _g():
                g_desc(nxt, nb).start()

    w_desc(NCHUNK - 1, (NCHUNK - 1) % NBUF).wait()


RB = 2048


def _repack_body(i_ref, o_ref):
    o_ref[...] = i_ref[:, :D_FEAT]


_repack_call = pl.pallas_call(
    _repack_body,
    grid=(B_TOKENS // RB,),
    in_specs=[pl.BlockSpec((RB, D_PAD), lambda i: (i, 0))],
    out_specs=pl.BlockSpec((RB, D_FEAT), lambda i: (i, 0)),
    out_shape=jax.ShapeDtypeStruct((B_TOKENS, D_FEAT), jnp.float32),
)


def kernel(x, embedding):
    mu, inv, sig, sel = _expansion_consts()
    table = _table_call(embedding, sel, mu, inv, sig)
    idx = x.reshape(-1).astype(jnp.int32)
    out = _gather_call(table, idx)
    return _repack_call(out).reshape(BATCH, SEQ, D_FEAT)


# speedup vs baseline: 1.1332x; 1.1332x over previous
import functools

import numpy as np
import jax
import jax.numpy as jnp
from jax import lax
from jax.experimental import pallas as pl
from jax.experimental.pallas import tpu as pltpu
from jax.experimental.pallas import tpu_sc as plsc

D_FEAT = 123
D_PAD = 128
NUM_VALS = 21
NC, NS = 2, 16
NW = NC * NS
BATCH, SEQ = 1024, 512
B_TOKENS = BATCH * SEQ
TOK_PER_W = B_TOKENS // NW
CHUNK = 64
NCHUNK = TOK_PER_W // CHUNK
NBUF = 4
LOOKAHEAD = 3


def _expansion_consts():
    mu = np.zeros((D_PAD,), np.float32)
    inv = np.zeros((D_PAD,), np.float32)
    ch = np.zeros((D_PAD,), np.int64)
    sig = np.zeros((D_PAD,), np.float32)
    o = 0
    for cnt, c, lo, hi, stride in (
        (90, 0, -4.5, 4.5, 0.1),
        (22, 1, 0.0, 2.2, 0.1),
        (8, 2, -1.0, 1.0, 0.25),
    ):
        mu[o:o + cnt] = np.linspace(lo, hi, cnt)
        inv[o:o + cnt] = 1.0 / stride
        ch[o:o + cnt] = c
        o += cnt
    for c in (3, 4, 5):
        ch[o] = c
        sig[o] = 1.0
        o += 1
    sel = np.zeros((6, D_PAD), np.float32)
    sel[ch[:D_FEAT], np.arange(D_FEAT)] = 1.0
    return (
        jnp.asarray(mu).reshape(1, -1),
        jnp.asarray(inv).reshape(1, -1),
        jnp.asarray(sig).reshape(1, -1),
        jnp.asarray(sel),
    )


def _table_body(emb_ref, sel_ref, mu_ref, inv_ref, sig_ref, tab_ref):
    e = jnp.zeros(tab_ref.shape, jnp.float32)
    for c in range(6):
        e = e + emb_ref[:, c:c + 1] * sel_ref[c:c + 1, :]
    rbf = jnp.exp(-((e - mu_ref[...]) * inv_ref[...]) ** 2)
    sg = jax.nn.sigmoid(e * 6.0 - 3.0)
    tab_ref[...] = jnp.where(sig_ref[...] > 0.0, sg, rbf)


_table_call = pl.pallas_call(
    _table_body,
    out_shape=jax.ShapeDtypeStruct((NUM_VALS, D_PAD), jnp.float32),
)


@functools.partial(
    pl.kernel,
    out_type=jax.ShapeDtypeStruct((B_TOKENS, D_FEAT), jnp.float32),
    mesh=plsc.VectorSubcoreMesh(core_axis_name="c", subcore_axis_name="s"),
    scratch_types=[
        pltpu.VMEM((TOK_PER_W,), jnp.int32),
        pltpu.VMEM_SHARED((NUM_VALS, D_PAD), jnp.float32),
        pltpu.VMEM((NBUF, CHUNK, D_PAD), jnp.float32),
        pltpu.VMEM((NBUF, CHUNK, D_FEAT), jnp.float32),
        pltpu.SemaphoreType.DMA,
        pltpu.SemaphoreType.DMA,
        pltpu.SemaphoreType.DMA,
    ],
)
def _gather_call(tab_hbm, idx_hbm, out_hbm, idx_v, tab_v, bufs, bufs123, tsem, gsem, wsem):
    sid = lax.axis_index("s")
    wid = sid * NC + lax.axis_index("c")
    base = wid * TOK_PER_W

    @pl.when(sid == 0)
    def _stage_table():
        pltpu.make_async_copy(tab_hbm, tab_v, tsem).start()

    pltpu.sync_copy(idx_hbm.at[pl.ds(base, TOK_PER_W)], idx_v)

    @pl.when(sid == 0)
    def _stage_table_wait():
        pltpu.make_async_copy(tab_hbm, tab_v, tsem).wait()

    plsc.subcore_barrier()

    def g_desc(c, b):
        return pltpu.make_async_copy(
            tab_v.at[idx_v.at[pl.ds(c * CHUNK, CHUNK)]], bufs.at[b], gsem)

    def w_desc(c, b):
        return pltpu.make_async_copy(
            bufs123.at[b], out_hbm.at[pl.ds(base + c * CHUNK, CHUNK)], wsem)

    for c in range(LOOKAHEAD):
        g_desc(c, c % NBUF).start()

    @pl.loop(0, NCHUNK, step=NBUF)
    def _ring(c4):
        for d in range(NBUF):
            c = c4 + d
            g_desc(c, d).wait()

            @pl.loop(0, CHUNK, unroll=4)
            def _squeeze(r):
                for k in range(7):
                    bufs123[d, r, pl.ds(k * 16, 16)] = bufs[d, r, pl.ds(k * 16, 16)]
                bufs123[d, r, pl.ds(D_FEAT - 16, 16)] = (
                    bufs[d, r, pl.ds(D_FEAT - 16, 16)])

            w_desc(c, d).start()
            prev = c - 1
            if d == 0:
                @pl.when(c4 > 0)
                def _w():
                    w_desc(prev, NBUF - 1).wait()
            else:
                w_desc(prev, d - 1).wait()
            nxt = c + LOOKAHEAD
            nb = (d + LOOKAHEAD) % NBUF

            @pl.when(nxt < NCHUNK)
            def _g():
                g_desc(nxt, nb).start()

    w_desc(NCHUNK - 1, (NCHUNK - 1) % NBUF).wait()


def kernel(x, embedding):
    mu, inv, sig, sel = _expansion_consts()
    table = _table_call(embedding, sel, mu, inv, sig)
    idx = x.reshape(-1).astype(jnp.int32)
    out = _gather_call(table, idx)
    return out.reshape(BATCH, SEQ, D_FEAT)


# speedup vs baseline: 1.8271x; 1.6123x over previous
import functools

import numpy as np
import jax
import jax.numpy as jnp
from jax import lax
from jax.experimental import pallas as pl
from jax.experimental.pallas import tpu as pltpu
from jax.experimental.pallas import tpu_sc as plsc

D_FEAT = 123
D_PAD = 128
NUM_VALS = 21
NC, NS = 2, 16
NW = NC * NS
BATCH, SEQ = 1024, 512
B_TOKENS = BATCH * SEQ
TOK_PER_W = B_TOKENS // NW
CHUNK = 128
NCHUNK = TOK_PER_W // CHUNK
NBUF = 4
LOOKAHEAD = 3


def _expansion_consts():
    mu = np.zeros((D_PAD,), np.float32)
    inv = np.zeros((D_PAD,), np.float32)
    ch = np.zeros((D_PAD,), np.int64)
    sig = np.zeros((D_PAD,), np.float32)
    o = 0
    for cnt, c, lo, hi, stride in (
        (90, 0, -4.5, 4.5, 0.1),
        (22, 1, 0.0, 2.2, 0.1),
        (8, 2, -1.0, 1.0, 0.25),
    ):
        mu[o:o + cnt] = np.linspace(lo, hi, cnt)
        inv[o:o + cnt] = 1.0 / stride
        ch[o:o + cnt] = c
        o += cnt
    for c in (3, 4, 5):
        ch[o] = c
        sig[o] = 1.0
        o += 1
    sel = np.zeros((6, D_PAD), np.float32)
    sel[ch[:D_FEAT], np.arange(D_FEAT)] = 1.0
    return (
        jnp.asarray(mu).reshape(1, -1),
        jnp.asarray(inv).reshape(1, -1),
        jnp.asarray(sig).reshape(1, -1),
        jnp.asarray(sel),
    )


def _table_body(emb_ref, sel_ref, mu_ref, inv_ref, sig_ref, tab_ref):
    e = jnp.zeros(tab_ref.shape, jnp.float32)
    for c in range(6):
        e = e + emb_ref[:, c:c + 1] * sel_ref[c:c + 1, :]
    rbf = jnp.exp(-((e - mu_ref[...]) * inv_ref[...]) ** 2)
    sg = jax.nn.sigmoid(e * 6.0 - 3.0)
    tab_ref[...] = jnp.where(sig_ref[...] > 0.0, sg, rbf)


_table_call = pl.pallas_call(
    _table_body,
    out_shape=jax.ShapeDtypeStruct((NUM_VALS, D_PAD), jnp.float32),
)


@functools.partial(
    pl.kernel,
    out_type=jax.ShapeDtypeStruct((B_TOKENS, D_PAD), jnp.float32),
    mesh=plsc.VectorSubcoreMesh(core_axis_name="c", subcore_axis_name="s"),
    scratch_types=[
        pltpu.VMEM((TOK_PER_W,), jnp.int32),
        pltpu.VMEM_SHARED((NUM_VALS, D_PAD), jnp.float32),
        pltpu.VMEM((NBUF, CHUNK, D_PAD), jnp.float32),
        pltpu.SemaphoreType.DMA,
        pltpu.SemaphoreType.DMA,
        pltpu.SemaphoreType.DMA,
    ],
)
def _gather_call(tab_hbm, idx_hbm, out_hbm, idx_v, tab_v, bufs, tsem, gsem, wsem):
    sid = lax.axis_index("s")
    wid = sid * NC + lax.axis_index("c")
    base = wid * TOK_PER_W

    @pl.when(sid == 0)
    def _stage_table():
        pltpu.make_async_copy(tab_hbm, tab_v, tsem).start()

    pltpu.sync_copy(idx_hbm.at[pl.ds(base, TOK_PER_W)], idx_v)

    @pl.when(sid == 0)
    def _stage_table_wait():
        pltpu.make_async_copy(tab_hbm, tab_v, tsem).wait()

    plsc.subcore_barrier()

    def g_desc(c, b):
        return pltpu.make_async_copy(
            tab_v.at[idx_v.at[pl.ds(c * CHUNK, CHUNK)]], bufs.at[b], gsem)

    def w_desc(c, b):
        return pltpu.make_async_copy(
            bufs.at[b], out_hbm.at[pl.ds(base + c * CHUNK, CHUNK)], wsem)

    for c in range(LOOKAHEAD):
        g_desc(c, c % NBUF).start()

    @pl.loop(0, NCHUNK, step=NBUF)
    def _ring(c4):
        for d in range(NBUF):
            c = c4 + d
            g_desc(c, d).wait()
            w_desc(c, d).start()
            prev = c - 1
            if d == 0:
                @pl.when(c4 > 0)
                def _w():
                    w_desc(prev, NBUF - 1).wait()
            else:
                w_desc(prev, d - 1).wait()
            nxt = c + LOOKAHEAD
            nb = (d + LOOKAHEAD) % NBUF

            @pl.when(nxt < NCHUNK)
            def _g():
                g_desc(nxt, nb).start()

    w_desc(NCHUNK - 1, (NCHUNK - 1) % NBUF).wait()


def kernel(x, embedding):
    mu, inv, sig, sel = _expansion_consts()
    table = _table_call(embedding, sel, mu, inv, sig)
    idx = x.reshape(-1).astype(jnp.int32)
    out = _gather_call(table, idx)
    return out[:, :D_FEAT].reshape(BATCH, SEQ, D_FEAT)
